# baseline (speedup 1.0000x reference)
import functools

import jax
import jax.numpy as jnp
from jax import lax
from jax.experimental import pallas as pl
from jax.experimental.pallas import tpu as pltpu
from jax.experimental.pallas import tpu_sc as plsc

DEG_COUNTS = [4000, 16000, 30000, 30000, 14000, 4000, 2000]
STARTS = [0, 4000, 20000, 50000, 80000, 94000, 98000]
N_ATOMS = 100000
BATCH = 1024
EPS = 1e-3

NC, NS = 2, 16
NW = NC * NS
MPW = BATCH // NW

CHUNKS = [(1, 200, 80), (2, 120, 250), (3, 80, 375),
          (4, 56, 250), (5, 40, 100), (6, 40, 50)]

CHUNKS_PL = [(1, 160, 100), (2, 120, 250), (3, 80, 375),
             (4, 56, 250), (5, 40, 100), (6, 40, 50)]

_MAXCH = [(n + NW - 1) // NW for _, _, n in CHUNKS]
ADJ_PAD = 128


def _chunk_range(wid, n):
    q, rem = divmod(n, NW)
    c0 = wid * q + jnp.minimum(wid, rem)
    my_n = q + (wid < rem).astype(jnp.int32)
    return c0, my_n

N_PAD = 102000
MEM_PAD = N_ATOMS + 256


def _sc_mesh():
    return plsc.VectorSubcoreMesh(core_axis_name="c", subcore_axis_name="s",
                                  num_cores=NC, num_subcores=NS)


def _wid():
    return lax.axis_index("s") * NC + lax.axis_index("c")


def _make_gather_sum(F, chunks=None):
    chunks = chunks or CHUNKS
    maxch = [(n + NW - 1) // NW for _, _, n in chunks]
    idx_scratch = [pltpu.VMEM((d * m * r,), jnp.int32)
                   for (d, r, _), m in zip(chunks, maxch)]

    @functools.partial(
        pl.kernel, mesh=_sc_mesh(),
        out_type=jax.ShapeDtypeStruct((N_ATOMS, F), jnp.float32),
        compiler_params=pltpu.CompilerParams(needs_layout_passes=False),
        scratch_types=idx_scratch + [
            pltpu.VMEM((3, 200, F), jnp.float32),
            pltpu.SemaphoreType.DMA,
        ] + [pltpu.SemaphoreType.DMA] * 9)
    def gather_sum(x_hbm, a1, a2, a3, a4, a5, a6, rel_hbm,
                   i1, i2, i3, i4, i5, i6, gb, sem_i, *sems):
        wid = _wid()
        sem_j0 = sems[0:3]
        sem_ad = sems[3:6]
        sem_co = sems[6:9]
        adjs = [a1, a2, a3, a4, a5, a6]
        idxs = [i1, i2, i3, i4, i5, i6]
        for (d, R, n), m, adj, idx_v in zip(chunks, maxch, adjs, idxs):
            start = STARTS[d]
            cd = DEG_COUNTS[d]
            c0, my_n = _chunk_range(wid, n)

            mr = m * R
            for j in range(d):
                pltpu.async_copy(adj.at[pl.ds(j * cd + c0 * R, mr)],
                                 idx_v.at[pl.ds(j * mr, mr)], sem_i)
            for j in range(d):
                pltpu.make_async_copy(adj.at[pl.ds(j * cd + c0 * R, mr)],
                                      idx_v.at[pl.ds(j * mr, mr)],
                                      sem_i).wait()

            def idx_slice(j, t, idx_v=idx_v, R=R, mr=mr):
                return idx_v.at[pl.ds(j * mr + t * R, R)]

            def gath(j, t, s, add, sem, x_hbm=x_hbm, R=R):
                return pltpu.make_async_copy(
                    x_hbm.at[idx_slice(j, t)], gb.at[s, pl.ds(0, R)], sem)

            def issue_j0(t, s):
                pltpu.async_copy(x_hbm.at[idx_slice(0, t)],
                                 gb.at[s, pl.ds(0, R)], sem_j0[s])

            def issue_adds(t, s, d=d):
                for j in range(1, d):
                    pltpu.async_copy(x_hbm.at[idx_slice(j, t)],
                                     gb.at[s, pl.ds(0, R)], sem_ad[s],
                                     add=True)

            def co_desc(t, s, R=R, start=start):
                return pltpu.make_async_copy(
                    gb.at[s, pl.ds(0, R)],
                    rel_hbm.at[pl.ds(start + (c0 + t) * R, R)], sem_co[s])

            n_grp = (my_n + 2) // 3

            @pl.loop(0, n_grp)
            def _(g, d=d, R=R, my_n=my_n):
                t0 = g * 3
                for s in range(3):
                    @pl.when((t0 + s < my_n) & (g > 0))
                    def _(t=t0 + s, s=s):
                        co_desc(t - 3, s).wait()

                    @pl.when(t0 + s < my_n)
                    def _(t=t0 + s, s=s):
                        issue_j0(t, s)
                if d > 1:
                    for s in range(3):
                        @pl.when(t0 + s < my_n)
                        def _(t=t0 + s, s=s):
                            gath(0, t, s, False, sem_j0[s]).wait()
                            issue_adds(t, s)
                    for s in range(3):
                        @pl.when(t0 + s < my_n)
                        def _(t=t0 + s, s=s, d=d):
                            for j in range(1, d):
                                gath(j, t, s, True, sem_ad[s]).wait()
                            co_desc(t, s).start()
                else:
                    for s in range(3):
                        @pl.when(t0 + s < my_n)
                        def _(t=t0 + s, s=s):
                            gath(0, t, s, False, sem_j0[s]).wait()
                            co_desc(t, s).start()

            for s in range(3):
                @pl.when(my_n > s)
                def _(s=s):
                    co_desc(0, s).wait()

    return gather_sum


def _make_pool(ow=128):
    F = 128
    chunks_pl = ([(1, 128, 125), (2, 80, 375), (3, 40, 750), (4, 40, 350),
                  (5, 32, 125), (6, 16, 125)] if ow == 64 else CHUNKS_PL)
    sr = max(r for _, r, _ in chunks_pl)
    ger = max(d * r for d, r, _ in chunks_pl)
    maxch = [(n + NW - 1) // NW for _, _, n in chunks_pl]
    idx_scratch = [pltpu.VMEM((d * m * r,), jnp.int32)
                   for (d, r, _), m in zip(chunks_pl, maxch)]
    nv = 4
    obuf_scratch = ([pltpu.VMEM((2, sr, 64), jnp.float32)] if ow == 64
                    else [])

    @functools.partial(
        pl.kernel, mesh=_sc_mesh(),
        out_type=jax.ShapeDtypeStruct((N_ATOMS, ow), jnp.float32),
        compiler_params=pltpu.CompilerParams(needs_layout_passes=False),
        scratch_types=idx_scratch + [
            pltpu.VMEM((2, sr, F), jnp.float32),
            pltpu.VMEM((2, ger, F), jnp.float32),
        ] + obuf_scratch + [
            pltpu.SemaphoreType.DMA,
        ] + [pltpu.SemaphoreType.DMA] * 4)
    def pool(x_hbm, a1, a2, a3, a4, a5, a6, out_hbm,
             i1, i2, i3, i4, i5, i6, sbuf, gbuf, *rest):
        if ow == 64:
            obuf, sem_i, *sems = rest
        else:
            sem_i, *sems = rest
            obuf = sbuf
        wid = _wid()
        sem_g = sems[0:2]
        sem_co = sems[2:4]
        r0 = wid * 128

        def deg0_copy(r0, nrows):
            pltpu.sync_copy(x_hbm.at[pl.ds(r0, nrows)],
                            sbuf.at[0, pl.ds(0, nrows)])
            if ow == 64:
                @pl.loop(0, nrows)
                def _(r):
                    for k in range(nv):
                        obuf[0, r, pl.ds(16 * k, 16)] = \
                            sbuf[0, r, pl.ds(16 * k, 16)]
            pltpu.sync_copy(obuf.at[0, pl.ds(0, nrows)],
                            out_hbm.at[pl.ds(r0, nrows)])

        @pl.when(wid < 31)
        def _():
            deg0_copy(wid * 128, 128)

        @pl.when(wid == 31)
        def _():
            deg0_copy(3968, 32)

        adjs = [a1, a2, a3, a4, a5, a6]
        idxs = [i1, i2, i3, i4, i5, i6]
        for (d, R, n), m, adj, idx_v in zip(chunks_pl, maxch, adjs, idxs):
            start = STARTS[d]
            cd = DEG_COUNTS[d]
            c0, my_n = _chunk_range(wid, n)

            mr = m * R
            for j in range(d):
                pltpu.async_copy(adj.at[pl.ds(j * cd + c0 * R, mr)],
                                 idx_v.at[pl.ds(j * mr, mr)], sem_i)
            for j in range(d):
                pltpu.make_async_copy(adj.at[pl.ds(j * cd + c0 * R, mr)],
                                      idx_v.at[pl.ds(j * mr, mr)],
                                      sem_i).wait()

            def self_desc(t, s, R=R, start=start, c0=c0):
                return pltpu.make_async_copy(
                    x_hbm.at[pl.ds(start + (c0 + t) * R, R)],
                    sbuf.at[s, pl.ds(0, R)], sem_g[s])

            def gath_desc(j, t, s, idx_v=idx_v, R=R, mr=mr):
                return pltpu.make_async_copy(
                    x_hbm.at[idx_v.at[pl.ds(j * mr + t * R, R)]],
                    gbuf.at[s, pl.ds(j * R, R)], sem_g[s])

            def co_desc(t, s, R=R, start=start, c0=c0):
                return pltpu.make_async_copy(
                    obuf.at[s, pl.ds(0, R)],
                    out_hbm.at[pl.ds(start + (c0 + t) * R, R)], sem_co[s])

            n_grp = (my_n + 1) // 2

            @pl.loop(0, n_grp)
            def _(g, d=d, R=R, my_n=my_n):
                t0 = g * 2
                for s in range(2):
                    @pl.when((t0 + s < my_n) & (g > 0))
                    def _(t=t0 + s, s=s):
                        co_desc(t - 2, s).wait()

                    @pl.when(t0 + s < my_n)
                    def _(t=t0 + s, s=s, d=d):
                        self_desc(t, s).start()
                        for j in range(d):
                            gath_desc(j, t, s).start()
                for s in range(2):
                    @pl.when(t0 + s < my_n)
                    def _(t=t0 + s, s=s, d=d, R=R):
                        self_desc(t, s).wait()
                        for j in range(d):
                            gath_desc(j, t, s).wait()

                        @pl.loop(0, R)
                        def _(r, d=d, R=R, s=s):
                            for k in range(nv):
                                acc = sbuf[s, r, pl.ds(16 * k, 16)]
                                for j in range(d):
                                    acc = jnp.maximum(
                                        acc, gbuf[s, j * R + r,
                                                  pl.ds(16 * k, 16)])
                                obuf[s, r, pl.ds(16 * k, 16)] = acc

                        co_desc(t, s).start()

            for s in range(2):
                @pl.when(my_n > s)
                def _(s=s):
                    co_desc(0, s).wait()

    return pool


def _make_segment():
    W = 256
    WM = W + 16

    @functools.partial(
        pl.kernel, mesh=_sc_mesh(),
        out_type=(jax.ShapeDtypeStruct((BATCH, 128), jnp.float32),
                  jax.ShapeDtypeStruct((BATCH, 128), jnp.float32)),
        compiler_params=pltpu.CompilerParams(needs_layout_passes=False),
        scratch_types=[
            pltpu.VMEM((2, W, 128), jnp.float32),
            pltpu.VMEM((2 * WM,), jnp.int32),
            pltpu.VMEM((32,), jnp.int32),
            pltpu.VMEM((MPW, 128), jnp.float32),
            pltpu.VMEM((MPW, 128), jnp.float32),
            pltpu.SemaphoreType.DMA,
            pltpu.SemaphoreType.DMA,
        ])
    def segment(h_hbm, mem_hbm, sums_hbm, maxs_hbm,
                win_h, win_m, probe, acc_s, acc_m, *sem_w):
        wid = _wid()
        m0 = wid * MPW

        def lower_bound(target):
            def body(_, lohi):
                lo, hi = lohi
                mid = (lo + hi) // 2
                al = (mid // 8) * 8
                pltpu.sync_copy(mem_hbm.at[pl.ds(al, 16)],
                                probe.at[pl.ds(0, 16)])
                v = probe[pl.ds(mid - al, 16)][0]
                go = lo < hi
                lt = v < target
                new_lo = jnp.where(go & lt, mid + 1, lo)
                new_hi = jnp.where(go & jnp.logical_not(lt), mid, hi)
                return new_lo, new_hi
            lo, _ = lax.fori_loop(0, 17, body, (jnp.int32(0),
                                                jnp.int32(N_ATOMS)))
            return lo

        rs = lower_bound(m0)
        re = lower_bound(m0 + MPW)
        ws0 = (rs // 8) * 8
        n_win = (re - ws0 + W - 1) // W

        def win_descs(t, s):
            ws = ws0 + W * t
            return (pltpu.make_async_copy(h_hbm.at[pl.ds(ws, W)],
                                          win_h.at[s], sem_w[s]),
                    pltpu.make_async_copy(mem_hbm.at[pl.ds(ws, W)],
                                          win_m.at[pl.ds(s * WM, W)],
                                          sem_w[s]))

        def issue(t, s):
            for de in win_descs(t, s):
                de.start()

        def drain(t, s):
            for de in win_descs(t, s):
                de.wait()

        zeros = jnp.zeros((16,), jnp.float32)
        ninf = jnp.full((16,), -jnp.inf, jnp.float32)
        init = (m0, (zeros,) * 8, (ninf,) * 8)

        def flush(cur_m, svs, mvs):
            slot = cur_m - m0
            for k in range(8):
                acc_s[slot, pl.ds(16 * k, 16)] = svs[k]
                acc_m[slot, pl.ds(16 * k, 16)] = mvs[k]

        @pl.when(n_win > 0)
        def _():
            issue(0, 0)

        @pl.when(n_win > 1)
        def _():
            issue(1, 1)

        def pair(g, carry):
            for s in range(2):
                t = 2 * g + s

                @pl.when(t < n_win)
                def _(t=t, s=s):
                    drain(t, s)

                ws = ws0 + W * t
                lo_r = jnp.maximum(rs - ws, 0)
                hi_r = jnp.maximum(lo_r, jnp.minimum(re - ws, W))

                def row(r, carry, s=s):
                    cur_m, svs, mvs = carry
                    m = win_m[pl.ds(s * WM + r, 16)][0]
                    new = m != cur_m

                    @pl.when(new)
                    def _():
                        flush(cur_m, svs, mvs)

                    nsvs, nmvs = [], []
                    for k in range(8):
                        h = win_h[s, r, pl.ds(16 * k, 16)]
                        nsvs.append(jnp.where(new, h, svs[k] + h))
                        nmvs.append(jnp.where(new, h, jnp.maximum(mvs[k], h)))
                    return m, tuple(nsvs), tuple(nmvs)

                carry = lax.fori_loop(lo_r, hi_r, row, carry)

                @pl.when(t + 2 < n_win)
                def _(t=t, s=s):
                    issue(t + 2, s)
            return carry

        n_pair = (n_win + 1) // 2
        cur_m, svs, mvs = pl.loop(0, n_pair, init_carry=init)(pair)
        flush(cur_m, svs, mvs)
        pltpu.sync_copy(acc_s, sums_hbm.at[pl.ds(m0, MPW)])
        pltpu.sync_copy(acc_m, maxs_hbm.at[pl.ds(m0, MPW)])

    return segment


BR = 2000
_DEG_BLOCK_BOUNDS = [2, 10, 25, 40, 47, 49]


def _deg_of(i):
    d = jnp.int32(0)
    for t in _DEG_BLOCK_BOUNDS:
        d = d + (i >= t).astype(jnp.int32)
    return d


def _conv_tc(x, rel, ws, wr, b, fin):
    def body(x_ref, r_ref, ws_ref, wr_ref, b_ref, o_ref):
        acc = jnp.dot(x_ref[:, :fin], ws_ref[0],
                      preferred_element_type=jnp.float32)
        acc = acc + jnp.dot(r_ref[:, :fin], wr_ref[0],
                            preferred_element_type=jnp.float32)
        o_ref[...] = jnp.concatenate(
            [jnp.tanh(acc + b_ref[0, 0]),
             jnp.zeros((BR, 64), jnp.float32)], axis=1)

    return pl.pallas_call(
        body,
        grid=(N_ATOMS // BR,),
        in_specs=[
            pl.BlockSpec((BR, 128), lambda i: (i, 0)),
            pl.BlockSpec((BR, 128), lambda i: (jnp.maximum(i, 2), 0)),
            pl.BlockSpec((1, fin, 64), lambda i: (_deg_of(i), 0, 0)),
            pl.BlockSpec((1, fin, 64), lambda i: (_deg_of(i), 0, 0)),
            pl.BlockSpec((1, 1, 64), lambda i: (_deg_of(i), 0, 0)),
        ],
        out_specs=pl.BlockSpec((BR, 128), lambda i: (i, 0)),
        out_shape=jax.ShapeDtypeStruct((N_ATOMS, 128), jnp.float32),
    )(x, rel, ws, wr, b)


def _dense1_tc(x, w, b, s3, b3):
    def body(x_ref, w_ref, b_ref, s_ref, b3_ref, o_ref):
        h = jnp.tanh(jnp.dot(x_ref[...], w_ref[...],
                             preferred_element_type=jnp.float32) + b_ref[0])
        o_ref[...] = h * s_ref[0] + b3_ref[0]

    return pl.pallas_call(
        body,
        grid=(N_PAD // BR,),
        in_specs=[
            pl.BlockSpec((BR, 64), lambda i: (jnp.minimum(i, 49), 0)),
            pl.BlockSpec((64, 128), lambda i: (0, 0)),
            pl.BlockSpec((1, 128), lambda i: (0, 0)),
            pl.BlockSpec((1, 128), lambda i: (0, 0)),
            pl.BlockSpec((1, 128), lambda i: (0, 0)),
        ],
        out_specs=pl.BlockSpec((BR, 128), lambda i: (i, 0)),
        out_shape=jax.ShapeDtypeStruct((N_PAD, 128), jnp.float32),
    )(x, w, b, s3, b3)


def _final_tc(sums, maxs, xa, w2a, w2b, w30, w3, const):
    def body(s_ref, m_ref, xa_ref, wa_ref, wb_ref, w30_ref, w3_ref,
             c_ref, o_ref):
        mv = jnp.dot(jnp.tanh(s_ref[...]), wa_ref[...],
                     preferred_element_type=jnp.float32)
        mv = mv + jnp.dot(jnp.tanh(m_ref[...]), wb_ref[...],
                          preferred_element_type=jnp.float32)
        extra = jnp.dot(xa_ref[...], w3_ref[...],
                        preferred_element_type=jnp.float32)
        o_ref[...] = mv * w30_ref[0, 0] + extra + c_ref[0, 0]

    return pl.pallas_call(
        body,
        out_shape=jax.ShapeDtypeStruct((BATCH, 1), jnp.float32),
    )(sums, maxs, xa, w2a, w2b, w30, w3, const)


def kernel(atom_features, deg_slice, membership, deg_adj_1, deg_adj_2,
           deg_adj_3, deg_adj_4, deg_adj_5, deg_adj_6, x_add, gc1_W, gc1_b,
           gc2_W, gc2_b, bn1_gamma, bn1_beta, bn1_mean, bn1_var, bn3_gamma,
           bn3_beta, bn3_mean, bn3_var, dense1_W, dense1_b, dense2_W,
           dense2_b, dense3_W, dense3_b):
    del deg_slice

    s1 = bn1_gamma * lax.rsqrt(bn1_var + EPS)

    def fold(W, b):
        ws = jnp.stack([W[12], W[0], W[2], W[4], W[6], W[8], W[10]])
        wr = jnp.stack([jnp.zeros_like(W[1]), W[1], W[3], W[5], W[7],
                        W[9], W[11]])
        bb = jnp.stack([b[6], b[0], b[1], b[2], b[3], b[4], b[5]])
        ws = ws * s1[None, None, :]
        wr = wr * s1[None, None, :]
        bb = (bb - bn1_mean[None, :]) * s1[None, :] + bn1_beta[None, :]
        return ws, wr, bb.reshape(7, 1, 64)

    ws1, wr1, b1 = fold(gc1_W, gc1_b)
    ws2, wr2, b2 = fold(gc2_W, gc2_b)

    s3 = (bn3_gamma * lax.rsqrt(bn3_var + EPS)).reshape(1, 128)
    b3f = (bn3_beta - bn3_mean * s3[0]).reshape(1, 128)

    pad = jnp.zeros((ADJ_PAD * 8,), jnp.int32)
    adjTs = [jnp.concatenate([a.T.reshape(-1), pad])
             for a in (deg_adj_1, deg_adj_2, deg_adj_3, deg_adj_4,
                       deg_adj_5, deg_adj_6)]
    mem_pad = jnp.concatenate(
        [membership, jnp.zeros((MEM_PAD - N_ATOMS,), jnp.int32)])
    xa_pad = jnp.concatenate(
        [jnp.zeros((BATCH, 1), jnp.float32), x_add], axis=1)
    w2a, w2b = dense2_W[:128], dense2_W[128:]
    w30 = dense3_W[0].reshape(1, 1)
    const = (dense2_b[0] * dense3_W[0, 0] + dense3_b[0]).reshape(1, 1)

    gather_sum = _make_gather_sum(128)
    rel1 = gather_sum(atom_features, *adjTs)
    h1 = _conv_tc(atom_features, rel1, ws1, wr1, b1, fin=128)
    h1p = _make_pool(128)(h1, *adjTs)
    rel2 = gather_sum(h1p, *adjTs)
    h2 = _conv_tc(h1p, rel2, ws2, wr2, b2, fin=64)
    h2p = _make_pool(64)(h2, *adjTs)
    h3 = _dense1_tc(h2p, dense1_W, dense1_b.reshape(1, 128), s3, b3f)
    sums, maxs = _make_segment()(h3, mem_pad)
    return _final_tc(sums, maxs, xa_pad, w2a, w2b, w30, dense3_W, const)


# speedup vs baseline: 1.0030x; 1.0030x over previous
import functools

import jax
import jax.numpy as jnp
from jax import lax
from jax.experimental import pallas as pl
from jax.experimental.pallas import tpu as pltpu
from jax.experimental.pallas import tpu_sc as plsc

DEG_COUNTS = [4000, 16000, 30000, 30000, 14000, 4000, 2000]
STARTS = [0, 4000, 20000, 50000, 80000, 94000, 98000]
N_ATOMS = 100000
BATCH = 1024
EPS = 1e-3

NC, NS = 2, 16
NW = NC * NS
MPW = BATCH // NW

CHUNKS = [(1, 200, 80), (2, 120, 250), (3, 80, 375),
          (4, 56, 250), (5, 40, 100), (6, 40, 50)]

CHUNKS_PL = [(1, 160, 100), (2, 120, 250), (3, 80, 375),
             (4, 56, 250), (5, 40, 100), (6, 40, 50)]

_MAXCH = [(n + NW - 1) // NW for _, _, n in CHUNKS]
ADJ_PAD = 128


def _chunk_range(wid, n):
    q, rem = divmod(n, NW)
    c0 = wid * q + jnp.minimum(wid, rem)
    my_n = q + (wid < rem).astype(jnp.int32)
    return c0, my_n

N_PAD = 102000
MEM_PAD = N_ATOMS + 256


def _sc_mesh():
    return plsc.VectorSubcoreMesh(core_axis_name="c", subcore_axis_name="s",
                                  num_cores=NC, num_subcores=NS)


def _wid():
    return lax.axis_index("s") * NC + lax.axis_index("c")


def _make_gather_sum(F):
    chunks = CHUNKS
    maxch = [(n + NW - 1) // NW for _, _, n in chunks]
    idx_scratch = [pltpu.VMEM((d * m * r,), jnp.int32)
                   for (d, r, _), m in zip(chunks, maxch)]

    @functools.partial(
        pl.kernel, mesh=_sc_mesh(),
        out_type=jax.ShapeDtypeStruct((N_ATOMS, F), jnp.float32),
        compiler_params=pltpu.CompilerParams(needs_layout_passes=False),
        scratch_types=idx_scratch + [
            pltpu.VMEM((3, 200, F), jnp.float32),
            pltpu.SemaphoreType.DMA,
        ] + [pltpu.SemaphoreType.DMA] * 9)
    def gather_sum(x_hbm, a1, a2, a3, a4, a5, a6, rel_hbm,
                   i1, i2, i3, i4, i5, i6, gb, sem_i, *sems):
        wid = _wid()
        sem_j0 = sems[0:3]
        sem_ad = sems[3:6]
        sem_co = sems[6:9]
        adjs = [a1, a2, a3, a4, a5, a6]
        idxs = [i1, i2, i3, i4, i5, i6]
        for (d, R, n), m, adj, idx_v in zip(chunks, maxch, adjs, idxs):
            start = STARTS[d]
            cd = DEG_COUNTS[d]
            c0, my_n = _chunk_range(wid, n)

            mr = m * R
            for j in range(d):
                pltpu.async_copy(adj.at[pl.ds(j * cd + c0 * R, mr)],
                                 idx_v.at[pl.ds(j * mr, mr)], sem_i)
            for j in range(d):
                pltpu.make_async_copy(adj.at[pl.ds(j * cd + c0 * R, mr)],
                                      idx_v.at[pl.ds(j * mr, mr)],
                                      sem_i).wait()

            def idx_slice(j, t, idx_v=idx_v, R=R, mr=mr):
                return idx_v.at[pl.ds(j * mr + t * R, R)]

            def gath(j, t, s, add, sem, x_hbm=x_hbm, R=R):
                return pltpu.make_async_copy(
                    x_hbm.at[idx_slice(j, t)], gb.at[s, pl.ds(0, R)], sem)

            def issue_j0(t, s):
                pltpu.async_copy(x_hbm.at[idx_slice(0, t)],
                                 gb.at[s, pl.ds(0, R)], sem_j0[s])

            def issue_adds(t, s, d=d):
                for j in range(1, d):
                    pltpu.async_copy(x_hbm.at[idx_slice(j, t)],
                                     gb.at[s, pl.ds(0, R)], sem_ad[s],
                                     add=True)

            def co_desc(t, s, R=R, start=start):
                return pltpu.make_async_copy(
                    gb.at[s, pl.ds(0, R)],
                    rel_hbm.at[pl.ds(start + (c0 + t) * R, R)], sem_co[s])

            n_grp = (my_n + 2) // 3

            @pl.loop(0, n_grp)
            def _(g, d=d, R=R, my_n=my_n):
                t0 = g * 3
                for s in range(3):
                    @pl.when((t0 + s < my_n) & (g > 0))
                    def _(t=t0 + s, s=s):
                        co_desc(t - 3, s).wait()

                    @pl.when(t0 + s < my_n)
                    def _(t=t0 + s, s=s):
                        issue_j0(t, s)
                if d > 1:
                    for s in range(3):
                        @pl.when(t0 + s < my_n)
                        def _(t=t0 + s, s=s):
                            gath(0, t, s, False, sem_j0[s]).wait()
                            issue_adds(t, s)
                    for s in range(3):
                        @pl.when(t0 + s < my_n)
                        def _(t=t0 + s, s=s, d=d):
                            for j in range(1, d):
                                gath(j, t, s, True, sem_ad[s]).wait()
                            co_desc(t, s).start()
                else:
                    for s in range(3):
                        @pl.when(t0 + s < my_n)
                        def _(t=t0 + s, s=s):
                            gath(0, t, s, False, sem_j0[s]).wait()
                            co_desc(t, s).start()

            for s in range(3):
                @pl.when(my_n > s)
                def _(s=s):
                    co_desc(0, s).wait()

    return gather_sum


def _make_pool():
    F = 128
    maxch = [(n + NW - 1) // NW for _, _, n in CHUNKS_PL]
    idx_scratch = [pltpu.VMEM((d * m * r,), jnp.int32)
                   for (d, r, _), m in zip(CHUNKS_PL, maxch)]
    nv = 4

    @functools.partial(
        pl.kernel, mesh=_sc_mesh(),
        out_type=jax.ShapeDtypeStruct((N_ATOMS, F), jnp.float32),
        compiler_params=pltpu.CompilerParams(needs_layout_passes=False),
        scratch_types=idx_scratch + [
            pltpu.VMEM((2, 160, F), jnp.float32),
            pltpu.VMEM((2, 240, F), jnp.float32),
            pltpu.SemaphoreType.DMA,
        ] + [pltpu.SemaphoreType.DMA] * 4)
    def pool(x_hbm, a1, a2, a3, a4, a5, a6, out_hbm,
             i1, i2, i3, i4, i5, i6, sbuf, gbuf, sem_i, *sems):
        obuf = sbuf
        wid = _wid()
        sem_g = sems[0:2]
        sem_co = sems[2:4]

        def deg0_copy(r0, nrows):
            pltpu.sync_copy(x_hbm.at[pl.ds(r0, nrows)],
                            sbuf.at[0, pl.ds(0, nrows)])
            pltpu.sync_copy(sbuf.at[0, pl.ds(0, nrows)],
                            out_hbm.at[pl.ds(r0, nrows)])

        @pl.when(wid < 31)
        def _():
            deg0_copy(wid * 128, 128)

        @pl.when(wid == 31)
        def _():
            deg0_copy(3968, 32)

        adjs = [a1, a2, a3, a4, a5, a6]
        idxs = [i1, i2, i3, i4, i5, i6]
        for (d, R, n), m, adj, idx_v in zip(CHUNKS_PL, maxch, adjs, idxs):
            start = STARTS[d]
            cd = DEG_COUNTS[d]
            c0, my_n = _chunk_range(wid, n)

            mr = m * R
            for j in range(d):
                pltpu.async_copy(adj.at[pl.ds(j * cd + c0 * R, mr)],
                                 idx_v.at[pl.ds(j * mr, mr)], sem_i)
            for j in range(d):
                pltpu.make_async_copy(adj.at[pl.ds(j * cd + c0 * R, mr)],
                                      idx_v.at[pl.ds(j * mr, mr)],
                                      sem_i).wait()

            def self_desc(t, s, R=R, start=start, c0=c0):
                return pltpu.make_async_copy(
                    x_hbm.at[pl.ds(start + (c0 + t) * R, R)],
                    sbuf.at[s, pl.ds(0, R)], sem_g[s])

            def gath_desc(j, t, s, idx_v=idx_v, R=R, mr=mr):
                return pltpu.make_async_copy(
                    x_hbm.at[idx_v.at[pl.ds(j * mr + t * R, R)]],
                    gbuf.at[s, pl.ds(j * R, R)], sem_g[s])

            def co_desc(t, s, R=R, start=start, c0=c0):
                return pltpu.make_async_copy(
                    obuf.at[s, pl.ds(0, R)],
                    out_hbm.at[pl.ds(start + (c0 + t) * R, R)], sem_co[s])

            n_grp = (my_n + 1) // 2

            @pl.loop(0, n_grp)
            def _(g, d=d, R=R, my_n=my_n):
                t0 = g * 2
                for s in range(2):
                    @pl.when((t0 + s < my_n) & (g > 0))
                    def _(t=t0 + s, s=s):
                        co_desc(t - 2, s).wait()

                    @pl.when(t0 + s < my_n)
                    def _(t=t0 + s, s=s, d=d):
                        self_desc(t, s).start()
                        for j in range(d):
                            gath_desc(j, t, s).start()
                for s in range(2):
                    @pl.when(t0 + s < my_n)
                    def _(t=t0 + s, s=s, d=d, R=R):
                        self_desc(t, s).wait()
                        for j in range(d):
                            gath_desc(j, t, s).wait()

                        @pl.loop(0, R)
                        def _(r, d=d, R=R, s=s):
                            for k in range(nv):
                                acc = sbuf[s, r, pl.ds(16 * k, 16)]
                                for j in range(d):
                                    acc = jnp.maximum(
                                        acc, gbuf[s, j * R + r,
                                                  pl.ds(16 * k, 16)])
                                obuf[s, r, pl.ds(16 * k, 16)] = acc

                        co_desc(t, s).start()

            for s in range(2):
                @pl.when(my_n > s)
                def _(s=s):
                    co_desc(0, s).wait()

    return pool


def _make_segment():
    W = 256
    WM = W + 16

    @functools.partial(
        pl.kernel, mesh=_sc_mesh(),
        out_type=(jax.ShapeDtypeStruct((BATCH, 128), jnp.float32),
                  jax.ShapeDtypeStruct((BATCH, 128), jnp.float32)),
        compiler_params=pltpu.CompilerParams(needs_layout_passes=False),
        scratch_types=[
            pltpu.VMEM((2, W, 128), jnp.float32),
            pltpu.VMEM((2 * WM,), jnp.int32),
            pltpu.VMEM((32,), jnp.int32),
            pltpu.VMEM((MPW, 128), jnp.float32),
            pltpu.VMEM((MPW, 128), jnp.float32),
            pltpu.SemaphoreType.DMA,
            pltpu.SemaphoreType.DMA,
        ])
    def segment(h_hbm, mem_hbm, sums_hbm, maxs_hbm,
                win_h, win_m, probe, acc_s, acc_m, *sem_w):
        wid = _wid()
        m0 = wid * MPW

        def lower_bound(target):
            def body(_, lohi):
                lo, hi = lohi
                mid = (lo + hi) // 2
                al = (mid // 8) * 8
                pltpu.sync_copy(mem_hbm.at[pl.ds(al, 16)],
                                probe.at[pl.ds(0, 16)])
                v = probe[pl.ds(mid - al, 16)][0]
                go = lo < hi
                lt = v < target
                new_lo = jnp.where(go & lt, mid + 1, lo)
                new_hi = jnp.where(go & jnp.logical_not(lt), mid, hi)
                return new_lo, new_hi
            lo, _ = lax.fori_loop(0, 17, body, (jnp.int32(0),
                                                jnp.int32(N_ATOMS)))
            return lo

        rs = lower_bound(m0)
        re = lower_bound(m0 + MPW)
        ws0 = (rs // 8) * 8
        n_win = (re - ws0 + W - 1) // W

        def win_descs(t, s):
            ws = ws0 + W * t
            return (pltpu.make_async_copy(h_hbm.at[pl.ds(ws, W)],
                                          win_h.at[s], sem_w[s]),
                    pltpu.make_async_copy(mem_hbm.at[pl.ds(ws, W)],
                                          win_m.at[pl.ds(s * WM, W)],
                                          sem_w[s]))

        def issue(t, s):
            for de in win_descs(t, s):
                de.start()

        def drain(t, s):
            for de in win_descs(t, s):
                de.wait()

        zeros = jnp.zeros((16,), jnp.float32)
        ninf = jnp.full((16,), -jnp.inf, jnp.float32)
        init = (m0, (zeros,) * 8, (ninf,) * 8)

        def flush(cur_m, svs, mvs):
            slot = cur_m - m0
            for k in range(8):
                acc_s[slot, pl.ds(16 * k, 16)] = svs[k]
                acc_m[slot, pl.ds(16 * k, 16)] = mvs[k]

        @pl.when(n_win > 0)
        def _():
            issue(0, 0)

        @pl.when(n_win > 1)
        def _():
            issue(1, 1)

        def pair(g, carry):
            for s in range(2):
                t = 2 * g + s

                @pl.when(t < n_win)
                def _(t=t, s=s):
                    drain(t, s)

                ws = ws0 + W * t
                lo_r = jnp.maximum(rs - ws, 0)
                hi_r = jnp.maximum(lo_r, jnp.minimum(re - ws, W))

                def row(r, carry, s=s):
                    cur_m, svs, mvs = carry
                    m = win_m[pl.ds(s * WM + r, 16)][0]
                    new = m != cur_m

                    @pl.when(new)
                    def _():
                        flush(cur_m, svs, mvs)

                    nsvs, nmvs = [], []
                    for k in range(8):
                        h = win_h[s, r, pl.ds(16 * k, 16)]
                        nsvs.append(jnp.where(new, h, svs[k] + h))
                        nmvs.append(jnp.where(new, h, jnp.maximum(mvs[k], h)))
                    return m, tuple(nsvs), tuple(nmvs)

                carry = lax.fori_loop(lo_r, hi_r, row, carry)

                @pl.when(t + 2 < n_win)
                def _(t=t, s=s):
                    issue(t + 2, s)
            return carry

        n_pair = (n_win + 1) // 2
        cur_m, svs, mvs = pl.loop(0, n_pair, init_carry=init)(pair)
        flush(cur_m, svs, mvs)
        pltpu.sync_copy(acc_s, sums_hbm.at[pl.ds(m0, MPW)])
        pltpu.sync_copy(acc_m, maxs_hbm.at[pl.ds(m0, MPW)])

    return segment


BR = 2000
_DEG_BLOCK_BOUNDS = [2, 10, 25, 40, 47, 49]


def _deg_of(i):
    d = jnp.int32(0)
    for t in _DEG_BLOCK_BOUNDS:
        d = d + (i >= t).astype(jnp.int32)
    return d


def _conv_tc(x, rel, ws, wr, b, fin):
    def body(x_ref, r_ref, ws_ref, wr_ref, b_ref, o_ref):
        acc = jnp.dot(x_ref[:, :fin], ws_ref[0],
                      preferred_element_type=jnp.float32)
        acc = acc + jnp.dot(r_ref[:, :fin], wr_ref[0],
                            preferred_element_type=jnp.float32)
        o_ref[...] = jnp.concatenate(
            [jnp.tanh(acc + b_ref[0, 0]),
             jnp.zeros((BR, 64), jnp.float32)], axis=1)

    return pl.pallas_call(
        body,
        grid=(N_ATOMS // BR,),
        in_specs=[
            pl.BlockSpec((BR, 128), lambda i: (i, 0)),
            pl.BlockSpec((BR, 128), lambda i: (jnp.maximum(i, 2), 0)),
            pl.BlockSpec((1, fin, 64), lambda i: (_deg_of(i), 0, 0)),
            pl.BlockSpec((1, fin, 64), lambda i: (_deg_of(i), 0, 0)),
            pl.BlockSpec((1, 1, 64), lambda i: (_deg_of(i), 0, 0)),
        ],
        out_specs=pl.BlockSpec((BR, 128), lambda i: (i, 0)),
        out_shape=jax.ShapeDtypeStruct((N_ATOMS, 128), jnp.float32),
    )(x, rel, ws, wr, b)


def _dense1_tc(x, w, b, s3, b3):
    def body(x_ref, w_ref, b_ref, s_ref, b3_ref, o_ref):
        h = jnp.tanh(jnp.dot(x_ref[:, :64], w_ref[...],
                             preferred_element_type=jnp.float32) + b_ref[0])
        o_ref[...] = h * s_ref[0] + b3_ref[0]

    return pl.pallas_call(
        body,
        grid=(N_PAD // BR,),
        in_specs=[
            pl.BlockSpec((BR, 128), lambda i: (jnp.minimum(i, 49), 0)),
            pl.BlockSpec((64, 128), lambda i: (0, 0)),
            pl.BlockSpec((1, 128), lambda i: (0, 0)),
            pl.BlockSpec((1, 128), lambda i: (0, 0)),
            pl.BlockSpec((1, 128), lambda i: (0, 0)),
        ],
        out_specs=pl.BlockSpec((BR, 128), lambda i: (i, 0)),
        out_shape=jax.ShapeDtypeStruct((N_PAD, 128), jnp.float32),
    )(x, w, b, s3, b3)


def _final_tc(sums, maxs, xa, w2a, w2b, w30, w3, const):
    def body(s_ref, m_ref, xa_ref, wa_ref, wb_ref, w30_ref, w3_ref,
             c_ref, o_ref):
        mv = jnp.dot(jnp.tanh(s_ref[...]), wa_ref[...],
                     preferred_element_type=jnp.float32)
        mv = mv + jnp.dot(jnp.tanh(m_ref[...]), wb_ref[...],
                          preferred_element_type=jnp.float32)
        extra = jnp.dot(xa_ref[...], w3_ref[...],
                        preferred_element_type=jnp.float32)
        o_ref[...] = mv * w30_ref[0, 0] + extra + c_ref[0, 0]

    return pl.pallas_call(
        body,
        out_shape=jax.ShapeDtypeStruct((BATCH, 1), jnp.float32),
    )(sums, maxs, xa, w2a, w2b, w30, w3, const)


def kernel(atom_features, deg_slice, membership, deg_adj_1, deg_adj_2,
           deg_adj_3, deg_adj_4, deg_adj_5, deg_adj_6, x_add, gc1_W, gc1_b,
           gc2_W, gc2_b, bn1_gamma, bn1_beta, bn1_mean, bn1_var, bn3_gamma,
           bn3_beta, bn3_mean, bn3_var, dense1_W, dense1_b, dense2_W,
           dense2_b, dense3_W, dense3_b):
    del deg_slice

    s1 = bn1_gamma * lax.rsqrt(bn1_var + EPS)

    def fold(W, b):
        ws = jnp.stack([W[12], W[0], W[2], W[4], W[6], W[8], W[10]])
        wr = jnp.stack([jnp.zeros_like(W[1]), W[1], W[3], W[5], W[7],
                        W[9], W[11]])
        bb = jnp.stack([b[6], b[0], b[1], b[2], b[3], b[4], b[5]])
        ws = ws * s1[None, None, :]
        wr = wr * s1[None, None, :]
        bb = (bb - bn1_mean[None, :]) * s1[None, :] + bn1_beta[None, :]
        return ws, wr, bb.reshape(7, 1, 64)

    ws1, wr1, b1 = fold(gc1_W, gc1_b)
    ws2, wr2, b2 = fold(gc2_W, gc2_b)

    s3 = (bn3_gamma * lax.rsqrt(bn3_var + EPS)).reshape(1, 128)
    b3f = (bn3_beta - bn3_mean * s3[0]).reshape(1, 128)

    pad = jnp.zeros((ADJ_PAD * 8,), jnp.int32)
    adjTs = [jnp.concatenate([a.T.reshape(-1), pad])
             for a in (deg_adj_1, deg_adj_2, deg_adj_3, deg_adj_4,
                       deg_adj_5, deg_adj_6)]
    mem_pad = jnp.concatenate(
        [membership, jnp.zeros((MEM_PAD - N_ATOMS,), jnp.int32)])
    xa_pad = jnp.concatenate(
        [jnp.zeros((BATCH, 1), jnp.float32), x_add], axis=1)
    w2a, w2b = dense2_W[:128], dense2_W[128:]
    w30 = dense3_W[0].reshape(1, 1)
    const = (dense2_b[0] * dense3_W[0, 0] + dense3_b[0]).reshape(1, 1)

    gather_sum = _make_gather_sum(128)
    pool = _make_pool()
    rel1 = gather_sum(atom_features, *adjTs)
    h1 = _conv_tc(atom_features, rel1, ws1, wr1, b1, fin=128)
    h1p = pool(h1, *adjTs)
    rel2 = gather_sum(h1p, *adjTs)
    h2 = _conv_tc(h1p, rel2, ws2, wr2, b2, fin=64)
    h2p = pool(h2, *adjTs)
    h3 = _dense1_tc(h2p, dense1_W, dense1_b.reshape(1, 128), s3, b3f)
    sums, maxs = _make_segment()(h3, mem_pad)
    return _final_tc(sums, maxs, xa_pad, w2a, w2b, w30, dense3_W, const)


# speedup vs baseline: 1.0136x; 1.0105x over previous
import functools

import jax
import jax.numpy as jnp
from jax import lax
from jax.experimental import pallas as pl
from jax.experimental.pallas import tpu as pltpu
from jax.experimental.pallas import tpu_sc as plsc

DEG_COUNTS = [4000, 16000, 30000, 30000, 14000, 4000, 2000]
STARTS = [0, 4000, 20000, 50000, 80000, 94000, 98000]
N_ATOMS = 100000
BATCH = 1024
EPS = 1e-3

NC, NS = 2, 16
NW = NC * NS
MPW = BATCH // NW

CHUNKS = [(1, 200, 80), (2, 120, 250), (3, 80, 375),
          (4, 56, 250), (5, 40, 100), (6, 40, 50)]

CHUNKS_PL = [(1, 160, 100), (2, 120, 250), (3, 80, 375),
             (4, 56, 250), (5, 40, 100), (6, 40, 50)]

_MAXCH = [(n + NW - 1) // NW for _, _, n in CHUNKS]
ADJ_PAD = 128


def _chunk_range(wid, n):
    q, rem = divmod(n, NW)
    c0 = wid * q + jnp.minimum(wid, rem)
    my_n = q + (wid < rem).astype(jnp.int32)
    return c0, my_n

N_PAD = 102000
MEM_PAD = N_ATOMS + 256


def _sc_mesh():
    return plsc.VectorSubcoreMesh(core_axis_name="c", subcore_axis_name="s",
                                  num_cores=NC, num_subcores=NS)


def _wid():
    return lax.axis_index("s") * NC + lax.axis_index("c")


def _make_gather_sum(F):
    chunks = CHUNKS_PL
    maxch = [(n + NW - 1) // NW for _, _, n in chunks]
    idx_scratch = [pltpu.VMEM((d * m * r,), jnp.int32)
                   for (d, r, _), m in zip(chunks, maxch)]

    @functools.partial(
        pl.kernel, mesh=_sc_mesh(),
        out_type=jax.ShapeDtypeStruct((N_ATOMS, F), jnp.float32),
        compiler_params=pltpu.CompilerParams(needs_layout_passes=False),
        scratch_types=idx_scratch + [
            pltpu.VMEM((4, 160, F), jnp.float32),
            pltpu.SemaphoreType.DMA,
        ] + [pltpu.SemaphoreType.DMA] * 12)
    def gather_sum(x_hbm, a1, a2, a3, a4, a5, a6, rel_hbm,
                   i1, i2, i3, i4, i5, i6, gb, sem_i, *sems):
        wid = _wid()
        sem_j0 = sems[0:4]
        sem_ad = sems[4:8]
        sem_co = sems[8:12]
        adjs = [a1, a2, a3, a4, a5, a6]
        idxs = [i1, i2, i3, i4, i5, i6]
        for (d, R, n), m, adj, idx_v in zip(chunks, maxch, adjs, idxs):
            start = STARTS[d]
            cd = DEG_COUNTS[d]
            c0, my_n = _chunk_range(wid, n)

            mr = m * R
            for j in range(d):
                pltpu.async_copy(adj.at[pl.ds(j * cd + c0 * R, mr)],
                                 idx_v.at[pl.ds(j * mr, mr)], sem_i)
            for j in range(d):
                pltpu.make_async_copy(adj.at[pl.ds(j * cd + c0 * R, mr)],
                                      idx_v.at[pl.ds(j * mr, mr)],
                                      sem_i).wait()

            def idx_slice(j, t, idx_v=idx_v, R=R, mr=mr):
                return idx_v.at[pl.ds(j * mr + t * R, R)]

            def gath(j, t, s, add, sem, x_hbm=x_hbm, R=R):
                return pltpu.make_async_copy(
                    x_hbm.at[idx_slice(j, t)], gb.at[s, pl.ds(0, R)], sem)

            def issue_j0(t, s):
                pltpu.async_copy(x_hbm.at[idx_slice(0, t)],
                                 gb.at[s, pl.ds(0, R)], sem_j0[s])

            def issue_adds(t, s, d=d):
                for j in range(1, d):
                    pltpu.async_copy(x_hbm.at[idx_slice(j, t)],
                                     gb.at[s, pl.ds(0, R)], sem_ad[s],
                                     add=True)

            def co_desc(t, s, R=R, start=start):
                return pltpu.make_async_copy(
                    gb.at[s, pl.ds(0, R)],
                    rel_hbm.at[pl.ds(start + (c0 + t) * R, R)], sem_co[s])

            n_grp = (my_n + 3) // 4

            @pl.loop(0, n_grp)
            def _(g, d=d, R=R, my_n=my_n):
                t0 = g * 4
                for s in range(4):
                    @pl.when((t0 + s < my_n) & (g > 0))
                    def _(t=t0 + s, s=s):
                        co_desc(t - 4, s).wait()

                    @pl.when(t0 + s < my_n)
                    def _(t=t0 + s, s=s):
                        issue_j0(t, s)
                if d > 1:
                    for s in range(4):
                        @pl.when(t0 + s < my_n)
                        def _(t=t0 + s, s=s):
                            gath(0, t, s, False, sem_j0[s]).wait()
                            issue_adds(t, s)
                    for s in range(4):
                        @pl.when(t0 + s < my_n)
                        def _(t=t0 + s, s=s, d=d):
                            for j in range(1, d):
                                gath(j, t, s, True, sem_ad[s]).wait()
                            co_desc(t, s).start()
                else:
                    for s in range(4):
                        @pl.when(t0 + s < my_n)
                        def _(t=t0 + s, s=s):
                            gath(0, t, s, False, sem_j0[s]).wait()
                            co_desc(t, s).start()

            for s in range(4):
                @pl.when(my_n > s)
                def _(s=s):
                    co_desc(0, s).wait()

    return gather_sum


def _make_pool():
    F = 128
    maxch = [(n + NW - 1) // NW for _, _, n in CHUNKS_PL]
    idx_scratch = [pltpu.VMEM((d * m * r,), jnp.int32)
                   for (d, r, _), m in zip(CHUNKS_PL, maxch)]
    nv = 4

    @functools.partial(
        pl.kernel, mesh=_sc_mesh(),
        out_type=jax.ShapeDtypeStruct((N_ATOMS, F), jnp.float32),
        compiler_params=pltpu.CompilerParams(needs_layout_passes=False),
        scratch_types=idx_scratch + [
            pltpu.VMEM((2, 160, F), jnp.float32),
            pltpu.VMEM((2, 240, F), jnp.float32),
            pltpu.SemaphoreType.DMA,
        ] + [pltpu.SemaphoreType.DMA] * 4)
    def pool(x_hbm, a1, a2, a3, a4, a5, a6, out_hbm,
             i1, i2, i3, i4, i5, i6, sbuf, gbuf, sem_i, *sems):
        obuf = sbuf
        wid = _wid()
        sem_g = sems[0:2]
        sem_co = sems[2:4]

        def deg0_copy(r0, nrows):
            pltpu.sync_copy(x_hbm.at[pl.ds(r0, nrows)],
                            sbuf.at[0, pl.ds(0, nrows)])
            pltpu.sync_copy(sbuf.at[0, pl.ds(0, nrows)],
                            out_hbm.at[pl.ds(r0, nrows)])

        @pl.when(wid < 31)
        def _():
            deg0_copy(wid * 128, 128)

        @pl.when(wid == 31)
        def _():
            deg0_copy(3968, 32)

        adjs = [a1, a2, a3, a4, a5, a6]
        idxs = [i1, i2, i3, i4, i5, i6]
        for (d, R, n), m, adj, idx_v in zip(CHUNKS_PL, maxch, adjs, idxs):
            start = STARTS[d]
            cd = DEG_COUNTS[d]
            c0, my_n = _chunk_range(wid, n)

            mr = m * R
            for j in range(d):
                pltpu.async_copy(adj.at[pl.ds(j * cd + c0 * R, mr)],
                                 idx_v.at[pl.ds(j * mr, mr)], sem_i)
            for j in range(d):
                pltpu.make_async_copy(adj.at[pl.ds(j * cd + c0 * R, mr)],
                                      idx_v.at[pl.ds(j * mr, mr)],
                                      sem_i).wait()

            def self_desc(t, s, R=R, start=start, c0=c0):
                return pltpu.make_async_copy(
                    x_hbm.at[pl.ds(start + (c0 + t) * R, R)],
                    sbuf.at[s, pl.ds(0, R)], sem_g[s])

            def gath_desc(j, t, s, idx_v=idx_v, R=R, mr=mr):
                return pltpu.make_async_copy(
                    x_hbm.at[idx_v.at[pl.ds(j * mr + t * R, R)]],
                    gbuf.at[s, pl.ds(j * R, R)], sem_g[s])

            def co_desc(t, s, R=R, start=start, c0=c0):
                return pltpu.make_async_copy(
                    obuf.at[s, pl.ds(0, R)],
                    out_hbm.at[pl.ds(start + (c0 + t) * R, R)], sem_co[s])

            n_grp = (my_n + 1) // 2

            @pl.loop(0, n_grp)
            def _(g, d=d, R=R, my_n=my_n):
                t0 = g * 2
                for s in range(2):
                    @pl.when((t0 + s < my_n) & (g > 0))
                    def _(t=t0 + s, s=s):
                        co_desc(t - 2, s).wait()

                    @pl.when(t0 + s < my_n)
                    def _(t=t0 + s, s=s, d=d):
                        self_desc(t, s).start()
                        for j in range(d):
                            gath_desc(j, t, s).start()
                for s in range(2):
                    @pl.when(t0 + s < my_n)
                    def _(t=t0 + s, s=s, d=d, R=R):
                        self_desc(t, s).wait()
                        for j in range(d):
                            gath_desc(j, t, s).wait()

                        @pl.loop(0, R)
                        def _(r, d=d, R=R, s=s):
                            for k in range(nv):
                                acc = sbuf[s, r, pl.ds(16 * k, 16)]
                                for j in range(d):
                                    acc = jnp.maximum(
                                        acc, gbuf[s, j * R + r,
                                                  pl.ds(16 * k, 16)])
                                obuf[s, r, pl.ds(16 * k, 16)] = acc

                        co_desc(t, s).start()

            for s in range(2):
                @pl.when(my_n > s)
                def _(s=s):
                    co_desc(0, s).wait()

    return pool


def _make_segment():
    W = 256
    WM = W + 16

    @functools.partial(
        pl.kernel, mesh=_sc_mesh(),
        out_type=(jax.ShapeDtypeStruct((BATCH, 128), jnp.float32),
                  jax.ShapeDtypeStruct((BATCH, 128), jnp.float32)),
        compiler_params=pltpu.CompilerParams(needs_layout_passes=False),
        scratch_types=[
            pltpu.VMEM((2, W, 128), jnp.float32),
            pltpu.VMEM((2 * WM,), jnp.int32),
            pltpu.VMEM((32,), jnp.int32),
            pltpu.VMEM((MPW, 128), jnp.float32),
            pltpu.VMEM((MPW, 128), jnp.float32),
            pltpu.SemaphoreType.DMA,
            pltpu.SemaphoreType.DMA,
        ])
    def segment(h_hbm, mem_hbm, sums_hbm, maxs_hbm,
                win_h, win_m, probe, acc_s, acc_m, *sem_w):
        wid = _wid()
        m0 = wid * MPW

        def lower_bound(target):
            def body(_, lohi):
                lo, hi = lohi
                mid = (lo + hi) // 2
                al = (mid // 8) * 8
                pltpu.sync_copy(mem_hbm.at[pl.ds(al, 16)],
                                probe.at[pl.ds(0, 16)])
                v = probe[pl.ds(mid - al, 16)][0]
                go = lo < hi
                lt = v < target
                new_lo = jnp.where(go & lt, mid + 1, lo)
                new_hi = jnp.where(go & jnp.logical_not(lt), mid, hi)
                return new_lo, new_hi
            lo, _ = lax.fori_loop(0, 17, body, (jnp.int32(0),
                                                jnp.int32(N_ATOMS)))
            return lo

        rs = lower_bound(m0)
        re = lower_bound(m0 + MPW)
        ws0 = (rs // 8) * 8
        n_win = (re - ws0 + W - 1) // W

        def win_descs(t, s):
            ws = ws0 + W * t
            return (pltpu.make_async_copy(h_hbm.at[pl.ds(ws, W)],
                                          win_h.at[s], sem_w[s]),
                    pltpu.make_async_copy(mem_hbm.at[pl.ds(ws, W)],
                                          win_m.at[pl.ds(s * WM, W)],
                                          sem_w[s]))

        def issue(t, s):
            for de in win_descs(t, s):
                de.start()

        def drain(t, s):
            for de in win_descs(t, s):
                de.wait()

        zeros = jnp.zeros((16,), jnp.float32)
        ninf = jnp.full((16,), -jnp.inf, jnp.float32)
        init = (m0, (zeros,) * 8, (ninf,) * 8)

        def flush(cur_m, svs, mvs):
            slot = cur_m - m0
            for k in range(8):
                acc_s[slot, pl.ds(16 * k, 16)] = svs[k]
                acc_m[slot, pl.ds(16 * k, 16)] = mvs[k]

        @pl.when(n_win > 0)
        def _():
            issue(0, 0)

        @pl.when(n_win > 1)
        def _():
            issue(1, 1)

        def pair(g, carry):
            for s in range(2):
                t = 2 * g + s

                @pl.when(t < n_win)
                def _(t=t, s=s):
                    drain(t, s)

                ws = ws0 + W * t
                lo_r = jnp.maximum(rs - ws, 0)
                hi_r = jnp.maximum(lo_r, jnp.minimum(re - ws, W))

                def row(r, carry, s=s):
                    cur_m, svs, mvs = carry
                    m = win_m[pl.ds(s * WM + r, 16)][0]
                    new = m != cur_m

                    @pl.when(new)
                    def _():
                        flush(cur_m, svs, mvs)

                    nsvs, nmvs = [], []
                    for k in range(8):
                        h = win_h[s, r, pl.ds(16 * k, 16)]
                        nsvs.append(jnp.where(new, h, svs[k] + h))
                        nmvs.append(jnp.where(new, h, jnp.maximum(mvs[k], h)))
                    return m, tuple(nsvs), tuple(nmvs)

                carry = lax.fori_loop(lo_r, hi_r, row, carry)

                @pl.when(t + 2 < n_win)
                def _(t=t, s=s):
                    issue(t + 2, s)
            return carry

        n_pair = (n_win + 1) // 2
        cur_m, svs, mvs = pl.loop(0, n_pair, init_carry=init)(pair)
        flush(cur_m, svs, mvs)
        pltpu.sync_copy(acc_s, sums_hbm.at[pl.ds(m0, MPW)])
        pltpu.sync_copy(acc_m, maxs_hbm.at[pl.ds(m0, MPW)])

    return segment


BR = 2000
_DEG_BLOCK_BOUNDS = [2, 10, 25, 40, 47, 49]


def _deg_of(i):
    d = jnp.int32(0)
    for t in _DEG_BLOCK_BOUNDS:
        d = d + (i >= t).astype(jnp.int32)
    return d


def _conv_tc(x, rel, ws, wr, b, fin):
    def body(x_ref, r_ref, ws_ref, wr_ref, b_ref, o_ref):
        acc = jnp.dot(x_ref[:, :fin], ws_ref[0],
                      preferred_element_type=jnp.float32)
        acc = acc + jnp.dot(r_ref[:, :fin], wr_ref[0],
                            preferred_element_type=jnp.float32)
        o_ref[...] = jnp.concatenate(
            [jnp.tanh(acc + b_ref[0, 0]),
             jnp.zeros((BR, 64), jnp.float32)], axis=1)

    return pl.pallas_call(
        body,
        grid=(N_ATOMS // BR,),
        in_specs=[
            pl.BlockSpec((BR, 128), lambda i: (i, 0)),
            pl.BlockSpec((BR, 128), lambda i: (jnp.maximum(i, 2), 0)),
            pl.BlockSpec((1, fin, 64), lambda i: (_deg_of(i), 0, 0)),
            pl.BlockSpec((1, fin, 64), lambda i: (_deg_of(i), 0, 0)),
            pl.BlockSpec((1, 1, 64), lambda i: (_deg_of(i), 0, 0)),
        ],
        out_specs=pl.BlockSpec((BR, 128), lambda i: (i, 0)),
        out_shape=jax.ShapeDtypeStruct((N_ATOMS, 128), jnp.float32),
    )(x, rel, ws, wr, b)


def _dense1_tc(x, w, b, s3, b3):
    def body(x_ref, w_ref, b_ref, s_ref, b3_ref, o_ref):
        h = jnp.tanh(jnp.dot(x_ref[:, :64], w_ref[...],
                             preferred_element_type=jnp.float32) + b_ref[0])
        o_ref[...] = h * s_ref[0] + b3_ref[0]

    return pl.pallas_call(
        body,
        grid=(N_PAD // BR,),
        in_specs=[
            pl.BlockSpec((BR, 128), lambda i: (jnp.minimum(i, 49), 0)),
            pl.BlockSpec((64, 128), lambda i: (0, 0)),
            pl.BlockSpec((1, 128), lambda i: (0, 0)),
            pl.BlockSpec((1, 128), lambda i: (0, 0)),
            pl.BlockSpec((1, 128), lambda i: (0, 0)),
        ],
        out_specs=pl.BlockSpec((BR, 128), lambda i: (i, 0)),
        out_shape=jax.ShapeDtypeStruct((N_PAD, 128), jnp.float32),
    )(x, w, b, s3, b3)


def _final_tc(sums, maxs, xa, w2a, w2b, w30, w3, const):
    def body(s_ref, m_ref, xa_ref, wa_ref, wb_ref, w30_ref, w3_ref,
             c_ref, o_ref):
        mv = jnp.dot(jnp.tanh(s_ref[...]), wa_ref[...],
                     preferred_element_type=jnp.float32)
        mv = mv + jnp.dot(jnp.tanh(m_ref[...]), wb_ref[...],
                          preferred_element_type=jnp.float32)
        extra = jnp.dot(xa_ref[...], w3_ref[...],
                        preferred_element_type=jnp.float32)
        o_ref[...] = mv * w30_ref[0, 0] + extra + c_ref[0, 0]

    return pl.pallas_call(
        body,
        out_shape=jax.ShapeDtypeStruct((BATCH, 1), jnp.float32),
    )(sums, maxs, xa, w2a, w2b, w30, w3, const)


def kernel(atom_features, deg_slice, membership, deg_adj_1, deg_adj_2,
           deg_adj_3, deg_adj_4, deg_adj_5, deg_adj_6, x_add, gc1_W, gc1_b,
           gc2_W, gc2_b, bn1_gamma, bn1_beta, bn1_mean, bn1_var, bn3_gamma,
           bn3_beta, bn3_mean, bn3_var, dense1_W, dense1_b, dense2_W,
           dense2_b, dense3_W, dense3_b):
    del deg_slice

    s1 = bn1_gamma * lax.rsqrt(bn1_var + EPS)

    def fold(W, b):
        ws = jnp.stack([W[12], W[0], W[2], W[4], W[6], W[8], W[10]])
        wr = jnp.stack([jnp.zeros_like(W[1]), W[1], W[3], W[5], W[7],
                        W[9], W[11]])
        bb = jnp.stack([b[6], b[0], b[1], b[2], b[3], b[4], b[5]])
        ws = ws * s1[None, None, :]
        wr = wr * s1[None, None, :]
        bb = (bb - bn1_mean[None, :]) * s1[None, :] + bn1_beta[None, :]
        return ws, wr, bb.reshape(7, 1, 64)

    ws1, wr1, b1 = fold(gc1_W, gc1_b)
    ws2, wr2, b2 = fold(gc2_W, gc2_b)

    s3 = (bn3_gamma * lax.rsqrt(bn3_var + EPS)).reshape(1, 128)
    b3f = (bn3_beta - bn3_mean * s3[0]).reshape(1, 128)

    pad = jnp.zeros((ADJ_PAD * 8,), jnp.int32)
    adjTs = [jnp.concatenate([a.T.reshape(-1), pad])
             for a in (deg_adj_1, deg_adj_2, deg_adj_3, deg_adj_4,
                       deg_adj_5, deg_adj_6)]
    mem_pad = jnp.concatenate(
        [membership, jnp.zeros((MEM_PAD - N_ATOMS,), jnp.int32)])
    xa_pad = jnp.concatenate(
        [jnp.zeros((BATCH, 1), jnp.float32), x_add], axis=1)
    w2a, w2b = dense2_W[:128], dense2_W[128:]
    w30 = dense3_W[0].reshape(1, 1)
    const = (dense2_b[0] * dense3_W[0, 0] + dense3_b[0]).reshape(1, 1)

    gather_sum = _make_gather_sum(128)
    pool = _make_pool()
    rel1 = gather_sum(atom_features, *adjTs)
    h1 = _conv_tc(atom_features, rel1, ws1, wr1, b1, fin=128)
    h1p = pool(h1, *adjTs)
    rel2 = gather_sum(h1p, *adjTs)
    h2 = _conv_tc(h1p, rel2, ws2, wr2, b2, fin=64)
    h2p = pool(h2, *adjTs)
    h3 = _dense1_tc(h2p, dense1_W, dense1_b.reshape(1, 128), s3, b3f)
    sums, maxs = _make_segment()(h3, mem_pad)
    return _final_tc(sums, maxs, xa_pad, w2a, w2b, w30, dense3_W, const)
